# baseline (speedup 1.0000x reference)
import functools

import jax
import jax.numpy as jnp
from jax.experimental import pallas as pl

_DH = 64
_G = 64
_BLK_MM = 10000
_BLK_POOL = 2048


def _emb_kernel(x_ref, we_ref, be_ref, wg_ref, o_ref):
    h = jnp.dot(x_ref[...], we_ref[...], preferred_element_type=jnp.float32)
    h = jnp.maximum(h + be_ref[...], 0.0)
    o_ref[...] = jnp.dot(h, wg_ref[...], preferred_element_type=jnp.float32)


def _mid_kernel(a_ref, b_ref, w_ref, o_ref):
    h = jnp.maximum(a_ref[...] + b_ref[...], 0.0)
    o_ref[...] = jnp.dot(h, w_ref[...], preferred_element_type=jnp.float32)


def _pool_kernel(a_ref, bg_ref, batch_ref, sums_ref, cnts_ref, maxs_ref):
    i = pl.program_id(0)

    @pl.when(i == 0)
    def _init():
        sums_ref[...] = jnp.zeros_like(sums_ref)
        cnts_ref[...] = jnp.zeros_like(cnts_ref)
        maxs_ref[...] = jnp.full_like(maxs_ref, -jnp.inf)

    h = jnp.maximum(a_ref[...] + bg_ref[...], 0.0)
    b = batch_ref[...]
    gids = jax.lax.broadcasted_iota(jnp.int32, (1, _G), 1)
    hit = b == gids
    onehot = hit.astype(jnp.float32)
    sums_ref[...] += jax.lax.dot_general(
        onehot, h, (((0,), (0,)), ((), ())),
        preferred_element_type=jnp.float32)
    ones = jnp.ones_like(h)
    cnts_ref[...] += jax.lax.dot_general(
        onehot, ones, (((0,), (0,)), ((), ())),
        preferred_element_type=jnp.float32)
    def body(g, carry):
        hm = jnp.where(b == g, h, -jnp.inf)
        row = jnp.max(hm, axis=0, keepdims=True)
        cur = maxs_ref[pl.ds(g, 1), :]
        maxs_ref[pl.ds(g, 1), :] = jnp.maximum(cur, row)
        return carry

    jax.lax.fori_loop(0, _G, body, 0)


def _head_kernel(sums_ref, cnts_ref, maxs_ref, wp_ref, bp_ref, wc_ref,
                 bc_ref, o_ref):
    mean = sums_ref[...] / jnp.maximum(cnts_ref[...], 1.0)
    hg = jnp.concatenate([mean, maxs_ref[...]], axis=1)
    hg = jnp.dot(hg, wp_ref[...], preferred_element_type=jnp.float32)
    hg = hg + bp_ref[...]
    logits = jnp.dot(hg, wc_ref[...], preferred_element_type=jnp.float32)
    logits = logits + bc_ref[...]
    mx = jnp.max(logits, axis=1, keepdims=True)
    shifted = logits - mx
    lse = jnp.log(jnp.sum(jnp.exp(shifted), axis=1, keepdims=True))
    o_ref[...] = shifted - lse


@jax.jit
def kernel(x, edge_index, batch, W_emb, b_emb, W_g1, b_g1, W_g2, b_g2,
           W_pool, b_pool, W_cls, b_cls):
    n, d_in = x.shape
    dh = W_emb.shape[1]
    c = W_cls.shape[1]
    grid_mm = n // _BLK_MM

    loop = jnp.arange(n, dtype=edge_index.dtype)
    src = jnp.concatenate([edge_index[0], loop])
    dst = jnp.concatenate([edge_index[1], loop])
    deg = jnp.zeros((n,), jnp.float32).at[dst].add(1.0)
    dinv = jax.lax.rsqrt(jnp.maximum(deg, 1e-12))
    norm = dinv[src] * dinv[dst]

    def rowblk(i):
        return (i, 0)

    def fixed(*_):
        return (0, 0)

    xw1 = pl.pallas_call(
        _emb_kernel,
        grid=(grid_mm,),
        in_specs=[
            pl.BlockSpec((_BLK_MM, d_in), rowblk),
            pl.BlockSpec((d_in, dh), fixed),
            pl.BlockSpec((1, dh), fixed),
            pl.BlockSpec((dh, dh), fixed),
        ],
        out_specs=pl.BlockSpec((_BLK_MM, dh), rowblk),
        out_shape=jax.ShapeDtypeStruct((n, dh), jnp.float32),
    )(x, W_emb, b_emb.reshape(1, dh), W_g1)

    agg1 = jnp.zeros((n, dh), jnp.float32).at[dst].add(
        xw1[src] * norm[:, None])

    xw2 = pl.pallas_call(
        _mid_kernel,
        grid=(grid_mm,),
        in_specs=[
            pl.BlockSpec((_BLK_MM, dh), rowblk),
            pl.BlockSpec((1, dh), fixed),
            pl.BlockSpec((dh, dh), fixed),
        ],
        out_specs=pl.BlockSpec((_BLK_MM, dh), rowblk),
        out_shape=jax.ShapeDtypeStruct((n, dh), jnp.float32),
    )(agg1, b_g1.reshape(1, dh), W_g2)

    agg2 = jnp.zeros((n, dh), jnp.float32).at[dst].add(
        xw2[src] * norm[:, None])

    grid_pool = -(-n // _BLK_POOL)
    n_pad = grid_pool * _BLK_POOL
    agg2_p = jnp.concatenate(
        [agg2, jnp.zeros((n_pad - n, dh), jnp.float32)], axis=0)
    batch_p = jnp.concatenate(
        [batch, jnp.full((n_pad - n,), _G, batch.dtype)]).reshape(n_pad, 1)

    sums, cnts, maxs = pl.pallas_call(
        _pool_kernel,
        grid=(grid_pool,),
        in_specs=[
            pl.BlockSpec((_BLK_POOL, dh), rowblk),
            pl.BlockSpec((1, dh), fixed),
            pl.BlockSpec((_BLK_POOL, 1), rowblk),
        ],
        out_specs=[
            pl.BlockSpec((_G, dh), fixed),
            pl.BlockSpec((_G, dh), fixed),
            pl.BlockSpec((_G, dh), fixed),
        ],
        out_shape=[
            jax.ShapeDtypeStruct((_G, dh), jnp.float32),
            jax.ShapeDtypeStruct((_G, dh), jnp.float32),
            jax.ShapeDtypeStruct((_G, dh), jnp.float32),
        ],
    )(agg2_p, b_g2.reshape(1, dh), batch_p)

    out = pl.pallas_call(
        _head_kernel,
        in_specs=[
            pl.BlockSpec((_G, dh), fixed),
            pl.BlockSpec((_G, dh), fixed),
            pl.BlockSpec((_G, dh), fixed),
            pl.BlockSpec((2 * dh, dh), fixed),
            pl.BlockSpec((1, dh), fixed),
            pl.BlockSpec((dh, c), fixed),
            pl.BlockSpec((1, c), fixed),
        ],
        out_specs=pl.BlockSpec((_G, c), fixed),
        out_shape=jax.ShapeDtypeStruct((_G, c), jnp.float32),
    )(sums, cnts, maxs, W_pool, b_pool.reshape(1, dh), W_cls,
      b_cls.reshape(1, c))
    return out
